# baseline (speedup 1.0000x reference)
import functools

import jax
import jax.numpy as jnp
from jax import lax
from jax.experimental import pallas as pl
from jax.experimental.pallas import tpu as pltpu
from jax.experimental.pallas import tpu_sc as plsc

N = 10000
E = 320000
D = 128
DE = 16
H = 256
EPS = 1e-7
BN_EPS = 1e-5

DH = D // 2
C = 64
NTILES = 16
NCH = 320
EPT = NCH * C
E_PAD = NTILES * EPT
NP = 10112
RPT = NP // NTILES
BE = 512
NEB = E_PAD // BE
BN = 1000


def _edge_mm_body(ea_ref, we_ref, o_ref):
    o_ref[...] = jnp.dot(ea_ref[...], we_ref[0],
                         preferred_element_type=jnp.float32)


def _edge_mm(ea_pad, We):
    we2 = We.reshape(DE, 2, DH).transpose(1, 0, 2)
    return pl.pallas_call(
        _edge_mm_body,
        grid=(2, NEB),
        in_specs=[
            pl.BlockSpec((BE, DE), lambda c, i: (i, 0)),
            pl.BlockSpec((1, DE, DH), lambda c, i: (c, 0, 0)),
        ],
        out_specs=pl.BlockSpec((BE, DH), lambda c, i: (c * NEB + i, 0)),
        out_shape=jax.ShapeDtypeStruct((2 * E_PAD, DH), jnp.float32),
    )(ea_pad, we2)


_sc_mesh = plsc.VectorSubcoreMesh(core_axis_name="c", subcore_axis_name="s",
                                  num_cores=2, num_subcores=16)


@functools.partial(
    pl.kernel,
    out_type=jax.ShapeDtypeStruct((2 * NP, D), jnp.float32),
    mesh=_sc_mesh,
    scratch_types=[
        pltpu.VMEM((4, 2 * C), jnp.int32),
        pltpu.VMEM((4, C), jnp.int32),
        pltpu.VMEM((2, C, D), jnp.float32),
        pltpu.VMEM((2, C, DH), jnp.float32),
        pltpu.VMEM((2, C, D), jnp.float32),
        pltpu.VMEM_SHARED((NP, D), jnp.float32),
        pltpu.SemaphoreType.DMA,
        pltpu.SemaphoreType.DMA,
        pltpu.SemaphoreType.DMA,
        pltpu.SemaphoreType.DMA,
        pltpu.SemaphoreType.DMA,
        pltpu.SemaphoreType.DMA,
        pltpu.SemaphoreType.DMA,
        pltpu.SemaphoreType.DMA,
        pltpu.SemaphoreType.DMA,
        pltpu.SemaphoreType.DMA,
    ],
)
def _sc_agg(x_h, e3, sd_h, out2,
            sdv, dstv, gbuf, ebuf, obuf, acc,
            sg0, sg1, se0, se1, sv0, sv1, sv2, sv3, so0, so1):
    c = lax.axis_index("c")
    s = lax.axis_index("s")
    zero = jnp.zeros((16,), jnp.float32)
    nsplat = jnp.full((16,), N, jnp.int32)
    sg = (sg0, sg1)
    se = (se0, se1)
    sv = (sv0, sv1, sv2, sv3)
    so = (so0, so1)

    def zrow(i, carry):
        for j in range(D // 16):
            obuf[0, i, pl.ds(j * 16, 16)] = zero
            obuf[1, i, pl.ds(j * 16, 16)] = zero
        return carry

    lax.fori_loop(0, C, zrow, 0)
    for k in range(9):
        pltpu.sync_copy(obuf.at[0], acc.at[pl.ds(s * RPT + k * C, C)])
    pltpu.sync_copy(obuf.at[0, pl.ds(0, RPT - 576)],
                    acc.at[pl.ds(s * RPT + 576, RPT - 576)])
    for j in range(C // 16):
        for d in range(4):
            dstv[d, pl.ds(j * 16, 16)] = nsplat
    plsc.subcore_barrier()
    pltpu.async_copy(obuf.at[0], acc.at[dstv.at[2]], so[0], add=True)
    pltpu.async_copy(obuf.at[1], acc.at[dstv.at[3]], so[1], add=True)

    ch = pl.multiple_of(c * DH, DH)
    cE = c * E_PAD
    base0 = s * NCH

    def sd_issue_d(k, d):
        kk = jnp.minimum(k, NCH - 1)
        pltpu.async_copy(sd_h.at[pl.ds((base0 + kk) * 2 * C, 2 * C)],
                         sdv.at[d], sv[d])

    def sd_wait_d(k, d):
        kk = jnp.minimum(k, NCH - 1)
        pltpu.make_async_copy(sd_h.at[pl.ds((base0 + kk) * 2 * C, 2 * C)],
                              sdv.at[d], sv[d]).wait()
        for j in range(C // 16):
            dstv[d, pl.ds(j * 16, 16)] = sdv[d, pl.ds(C + j * 16, 16)]

    def ge_issue_db(k, d, b):
        kk = jnp.minimum(k, NCH - 1)
        geb = (base0 + kk) * C
        pltpu.async_copy(x_h.at[sdv.at[d, pl.ds(0, C)]], gbuf.at[b], sg[b])
        pltpu.async_copy(e3.at[pl.ds(cE + geb, C)], ebuf.at[b], se[b])

    def ge_wait_db(k, d, b):
        kk = jnp.minimum(k, NCH - 1)
        geb = (base0 + kk) * C
        pltpu.make_async_copy(x_h.at[sdv.at[d, pl.ds(0, C)]],
                              gbuf.at[b], sg[b]).wait()
        pltpu.make_async_copy(e3.at[pl.ds(cE + geb, C)],
                              ebuf.at[b], se[b]).wait()

    def do_chunk(d, b):
        pltpu.make_async_copy(obuf.at[b], acc.at[dstv.at[d]], so[b]).wait()

        @plsc.parallel_loop(0, C, step=1, unroll=4)
        def rowfn(r):
            for j in range(DH // 16):
                sl = pl.ds(ch + j * 16, 16)
                m = jnp.maximum(gbuf[b, r, sl] + ebuf[b, r, pl.ds(j * 16, 16)],
                                0.0) + EPS
                t = jnp.exp(m)
                obuf[b, r, pl.ds(j * 16, 16)] = t
                obuf[b, r, pl.ds(DH + j * 16, 16)] = m * t

        pltpu.async_copy(obuf.at[b], acc.at[dstv.at[d]], so[b], add=True)

    sd_issue_d(0, 0)
    sd_issue_d(1, 1)
    sd_wait_d(0, 0)
    ge_issue_db(0, 0, 0)

    def quad(q, carry):
        k0 = 4 * q
        for b4 in range(4):
            k = k0 + b4
            b = b4 % 2
            sd_wait_d(k + 1, (b4 + 1) % 4)
            ge_issue_db(k + 1, (b4 + 1) % 4, 1 - b)
            sd_issue_d(k + 2, (b4 + 2) % 4)
            ge_wait_db(k, b4 % 4, b)
            do_chunk(b4 % 4, b)
        return carry

    lax.fori_loop(0, NCH // 4, quad, 0)
    sd_wait_d(NCH, 1)
    ge_wait_db(NCH, 0, 0)
    pltpu.make_async_copy(obuf.at[0], acc.at[dstv.at[2]], so[0]).wait()
    pltpu.make_async_copy(obuf.at[1], acc.at[dstv.at[3]], so[1]).wait()
    plsc.subcore_barrier()

    for k in range(9):
        off = s * RPT + k * C
        pltpu.sync_copy(acc.at[pl.ds(off, C)], obuf.at[0])
        pltpu.sync_copy(obuf.at[0], out2.at[pl.ds(c * NP + off, C)])
    off = s * RPT + 576
    vb = obuf.at[0, pl.ds(0, RPT - 576)]
    pltpu.sync_copy(acc.at[pl.ds(off, RPT - 576)], vb)
    pltpu.sync_copy(vb, out2.at[pl.ds(c * NP + off, RPT - 576)])


def _node_mlp_body(relu_out, o2a_ref, o2b_ref, x_ref, wa_ref, s1_ref, b1_ref,
                   wb_ref, o_ref):
    a = o2a_ref[0]
    b = o2b_ref[0]
    den = jnp.concatenate([a[:, :DH], b[:, :DH]], axis=1)
    num = jnp.concatenate([a[:, DH:], b[:, DH:]], axis=1)
    agg = num / jnp.where(den == 0.0, 1.0, den)
    o = agg + x_ref[...]
    h = jnp.dot(o, wa_ref[...], preferred_element_type=jnp.float32)
    h = jnp.maximum(h * s1_ref[...] + b1_ref[...], 0.0)
    y = jnp.dot(h, wb_ref[...], preferred_element_type=jnp.float32)
    if relu_out:
        y = jnp.maximum(y, 0.0)
    o_ref[...] = y


def _node_mlp(out2, x, Wa, s1, b1, Wb, relu_out):
    out2v = out2.reshape(2, NP, D)
    return pl.pallas_call(
        functools.partial(_node_mlp_body, relu_out),
        grid=(N // BN,),
        in_specs=[
            pl.BlockSpec((1, BN, D), lambda i: (0, i, 0)),
            pl.BlockSpec((1, BN, D), lambda i: (1, i, 0)),
            pl.BlockSpec((BN, D), lambda i: (i, 0)),
            pl.BlockSpec((D, H), lambda i: (0, 0)),
            pl.BlockSpec((1, H), lambda i: (0, 0)),
            pl.BlockSpec((1, H), lambda i: (0, 0)),
            pl.BlockSpec((H, D), lambda i: (0, 0)),
        ],
        out_specs=pl.BlockSpec((BN, D), lambda i: (i, 0)),
        out_shape=jax.ShapeDtypeStruct((N, D), jnp.float32),
    )(out2v, out2v, x, Wa, s1, b1, Wb)


def _layer(xin, sd, ea_pad, We, Wa, bnw, bnb, Wb, relu_out):
    e3 = _edge_mm(ea_pad, We)
    out2 = _sc_agg(xin, e3, sd)
    s1 = (bnw / jnp.sqrt(1.0 + BN_EPS)).reshape(1, H)
    b1 = bnb.reshape(1, H)
    return _node_mlp(out2, xin, Wa, s1, b1, Wb, relu_out)


def kernel(x, edge_index, edge_attr, We1, W1a, bn1w, bn1b, W1b,
           We2, W2a, bn2w, bn2b, W2b):
    pad = E_PAD - E
    src = jnp.concatenate([edge_index[0], jnp.zeros((pad,), jnp.int32)])
    dst = jnp.concatenate([edge_index[1], jnp.full((pad,), N, jnp.int32)])
    sd = jnp.concatenate(
        [src.reshape(-1, C), dst.reshape(-1, C)], axis=1).reshape(-1)
    ea_pad = jnp.concatenate(
        [edge_attr, jnp.zeros((pad, DE), jnp.float32)], axis=0)
    h = _layer(x, sd, ea_pad, We1, W1a, bn1w, bn1b, W1b, True)
    return _layer(h, sd, ea_pad, We2, W2a, bn2w, bn2b, W2b, False)


# speedup vs baseline: 1.1100x; 1.1100x over previous
import functools

import jax
import jax.numpy as jnp
from jax import lax
from jax.experimental import pallas as pl
from jax.experimental.pallas import tpu as pltpu
from jax.experimental.pallas import tpu_sc as plsc

N = 10000
E = 320000
D = 128
DE = 16
H = 256
EPS = 1e-7
BN_EPS = 1e-5

DH = D // 2
C = 64
NTILES = 16
NCH = 320
EPT = NCH * C
E_PAD = NTILES * EPT
NP = 10112
RPT = NP // NTILES
BE = 512
NEB = E_PAD // BE
BN = 1000


def _edge_mm_body(ea_ref, we_ref, o_ref):
    o_ref[...] = jnp.dot(ea_ref[...], we_ref[0],
                         preferred_element_type=jnp.float32)


def _edge_mm(ea_pad, We):
    we2 = We.reshape(DE, 2, DH).transpose(1, 0, 2)
    return pl.pallas_call(
        _edge_mm_body,
        grid=(2, NEB),
        in_specs=[
            pl.BlockSpec((BE, DE), lambda c, i: (i, 0)),
            pl.BlockSpec((1, DE, DH), lambda c, i: (c, 0, 0)),
        ],
        out_specs=pl.BlockSpec((BE, DH), lambda c, i: (c * NEB + i, 0)),
        out_shape=jax.ShapeDtypeStruct((2 * E_PAD, DH), jnp.float32),
    )(ea_pad, we2)


_sc_mesh = plsc.VectorSubcoreMesh(core_axis_name="c", subcore_axis_name="s",
                                  num_cores=2, num_subcores=16)


@functools.partial(
    pl.kernel,
    out_type=jax.ShapeDtypeStruct((2 * NP, D), jnp.float32),
    mesh=_sc_mesh,
    scratch_types=[
        pltpu.VMEM((4, 2 * C), jnp.int32),
        pltpu.VMEM((4, C), jnp.int32),
        pltpu.VMEM((2, C, D), jnp.float32),
        pltpu.VMEM((2, C, DH), jnp.float32),
        pltpu.VMEM((2, C, D), jnp.float32),
        pltpu.VMEM_SHARED((NP, D), jnp.float32),
        pltpu.SemaphoreType.DMA,
        pltpu.SemaphoreType.DMA,
        pltpu.SemaphoreType.DMA,
        pltpu.SemaphoreType.DMA,
        pltpu.SemaphoreType.DMA,
        pltpu.SemaphoreType.DMA,
        pltpu.SemaphoreType.DMA,
        pltpu.SemaphoreType.DMA,
        pltpu.SemaphoreType.DMA,
        pltpu.SemaphoreType.DMA,
    ],
)
def _sc_agg(x_h, e3, sd_h, out2,
            sdv, dstv, gbuf, ebuf, obuf, acc,
            sg0, sg1, se0, se1, sv0, sv1, sv2, sv3, so0, so1):
    c = lax.axis_index("c")
    s = lax.axis_index("s")
    zero = jnp.zeros((16,), jnp.float32)
    nsplat = jnp.full((16,), N, jnp.int32)
    sg = (sg0, sg1)
    se = (se0, se1)
    sv = (sv0, sv1, sv2, sv3)
    so = (so0, so1)

    def zrow(i, carry):
        for j in range(D // 16):
            obuf[0, i, pl.ds(j * 16, 16)] = zero
            obuf[1, i, pl.ds(j * 16, 16)] = zero
        return carry

    lax.fori_loop(0, C, zrow, 0)
    for k in range(9):
        pltpu.sync_copy(obuf.at[0], acc.at[pl.ds(s * RPT + k * C, C)])
    pltpu.sync_copy(obuf.at[0, pl.ds(0, RPT - 576)],
                    acc.at[pl.ds(s * RPT + 576, RPT - 576)])
    for j in range(C // 16):
        for d in range(4):
            dstv[d, pl.ds(j * 16, 16)] = nsplat
    plsc.subcore_barrier()
    pltpu.async_copy(obuf.at[0], acc.at[dstv.at[2]], so[0], add=True)
    pltpu.async_copy(obuf.at[1], acc.at[dstv.at[3]], so[1], add=True)

    cN = c * N
    cE = c * E_PAD
    base0 = s * NCH

    def sd_issue_d(k, d):
        kk = jnp.minimum(k, NCH - 1)
        pltpu.async_copy(sd_h.at[pl.ds((base0 + kk) * 2 * C, 2 * C)],
                         sdv.at[d], sv[d])

    def sd_wait_d(k, d):
        kk = jnp.minimum(k, NCH - 1)
        pltpu.make_async_copy(sd_h.at[pl.ds((base0 + kk) * 2 * C, 2 * C)],
                              sdv.at[d], sv[d]).wait()
        for j in range(C // 16):
            dstv[d, pl.ds(j * 16, 16)] = sdv[d, pl.ds(C + j * 16, 16)]
            sdv[d, pl.ds(j * 16, 16)] = sdv[d, pl.ds(j * 16, 16)] + cN

    def ge_issue_db(k, d, b):
        kk = jnp.minimum(k, NCH - 1)
        geb = (base0 + kk) * C
        pltpu.async_copy(x_h.at[sdv.at[d, pl.ds(0, C)]], gbuf.at[b], sg[b])
        pltpu.async_copy(e3.at[pl.ds(cE + geb, C)], ebuf.at[b], se[b])

    def ge_wait_db(k, d, b):
        kk = jnp.minimum(k, NCH - 1)
        geb = (base0 + kk) * C
        pltpu.make_async_copy(x_h.at[sdv.at[d, pl.ds(0, C)]],
                              gbuf.at[b], sg[b]).wait()
        pltpu.make_async_copy(e3.at[pl.ds(cE + geb, C)],
                              ebuf.at[b], se[b]).wait()

    def do_chunk(d, b):
        pltpu.make_async_copy(obuf.at[b], acc.at[dstv.at[d]], so[b]).wait()

        @plsc.parallel_loop(0, C, step=1, unroll=4)
        def rowfn(r):
            for j in range(DH // 16):
                sl = pl.ds(j * 16, 16)
                m = jnp.maximum(gbuf[b, r, sl] + ebuf[b, r, sl], 0.0) + EPS
                t = jnp.exp(m)
                obuf[b, r, pl.ds(j * 16, 16)] = t
                obuf[b, r, pl.ds(DH + j * 16, 16)] = m * t

        pltpu.async_copy(obuf.at[b], acc.at[dstv.at[d]], so[b], add=True)

    sd_issue_d(0, 0)
    sd_issue_d(1, 1)
    sd_wait_d(0, 0)
    ge_issue_db(0, 0, 0)

    def quad(q, carry):
        k0 = 4 * q
        for b4 in range(4):
            k = k0 + b4
            b = b4 % 2
            sd_wait_d(k + 1, (b4 + 1) % 4)
            ge_issue_db(k + 1, (b4 + 1) % 4, 1 - b)
            sd_issue_d(k + 2, (b4 + 2) % 4)
            ge_wait_db(k, b4 % 4, b)
            do_chunk(b4 % 4, b)
        return carry

    lax.fori_loop(0, NCH // 4, quad, 0)
    sd_wait_d(NCH, 1)
    ge_wait_db(NCH, 0, 0)
    pltpu.make_async_copy(obuf.at[0], acc.at[dstv.at[2]], so[0]).wait()
    pltpu.make_async_copy(obuf.at[1], acc.at[dstv.at[3]], so[1]).wait()
    plsc.subcore_barrier()

    for k in range(9):
        off = s * RPT + k * C
        pltpu.sync_copy(acc.at[pl.ds(off, C)], obuf.at[0])
        pltpu.sync_copy(obuf.at[0], out2.at[pl.ds(c * NP + off, C)])
    off = s * RPT + 576
    vb = obuf.at[0, pl.ds(0, RPT - 576)]
    pltpu.sync_copy(acc.at[pl.ds(off, RPT - 576)], vb)
    pltpu.sync_copy(vb, out2.at[pl.ds(c * NP + off, RPT - 576)])


def _node_mlp_body(relu_out, o2a_ref, o2b_ref, x_ref, wa_ref, s1_ref, b1_ref,
                   wb_ref, o_ref):
    a = o2a_ref[0]
    b = o2b_ref[0]
    den = jnp.concatenate([a[:, :DH], b[:, :DH]], axis=1)
    num = jnp.concatenate([a[:, DH:], b[:, DH:]], axis=1)
    agg = num / jnp.where(den == 0.0, 1.0, den)
    o = agg + x_ref[...]
    h = jnp.dot(o, wa_ref[...], preferred_element_type=jnp.float32)
    h = jnp.maximum(h * s1_ref[...] + b1_ref[...], 0.0)
    y = jnp.dot(h, wb_ref[...], preferred_element_type=jnp.float32)
    if relu_out:
        y = jnp.maximum(y, 0.0)
    o_ref[...] = y


def _node_mlp(out2, x, Wa, s1, b1, Wb, relu_out):
    out2v = out2.reshape(2, NP, D)
    return pl.pallas_call(
        functools.partial(_node_mlp_body, relu_out),
        grid=(N // BN,),
        in_specs=[
            pl.BlockSpec((1, BN, D), lambda i: (0, i, 0)),
            pl.BlockSpec((1, BN, D), lambda i: (1, i, 0)),
            pl.BlockSpec((BN, D), lambda i: (i, 0)),
            pl.BlockSpec((D, H), lambda i: (0, 0)),
            pl.BlockSpec((1, H), lambda i: (0, 0)),
            pl.BlockSpec((1, H), lambda i: (0, 0)),
            pl.BlockSpec((H, D), lambda i: (0, 0)),
        ],
        out_specs=pl.BlockSpec((BN, D), lambda i: (i, 0)),
        out_shape=jax.ShapeDtypeStruct((N, D), jnp.float32),
    )(out2v, out2v, x, Wa, s1, b1, Wb)


def _layer(xin, sd, ea_pad, We, Wa, bnw, bnb, Wb, relu_out):
    e3 = _edge_mm(ea_pad, We)
    x3 = jnp.concatenate(
        [jnp.concatenate([xin[:, :DH], xin[:, DH:]], axis=0),
         jnp.zeros((2 * N, DH), jnp.float32)], axis=1)
    out2 = _sc_agg(x3, e3, sd)
    s1 = (bnw / jnp.sqrt(1.0 + BN_EPS)).reshape(1, H)
    b1 = bnb.reshape(1, H)
    return _node_mlp(out2, xin, Wa, s1, b1, Wb, relu_out)


def kernel(x, edge_index, edge_attr, We1, W1a, bn1w, bn1b, W1b,
           We2, W2a, bn2w, bn2b, W2b):
    pad = E_PAD - E
    src = jnp.concatenate([edge_index[0], jnp.zeros((pad,), jnp.int32)])
    dst = jnp.concatenate([edge_index[1], jnp.full((pad,), N, jnp.int32)])
    sd = jnp.concatenate(
        [src.reshape(-1, C), dst.reshape(-1, C)], axis=1).reshape(-1)
    ea_pad = jnp.concatenate(
        [edge_attr, jnp.zeros((pad, DE), jnp.float32)], axis=0)
    h = _layer(x, sd, ea_pad, We1, W1a, bn1w, bn1b, W1b, True)
    return _layer(h, sd, ea_pad, We2, W2a, bn2w, bn2b, W2b, False)


# speedup vs baseline: 1.1745x; 1.0580x over previous
import functools

import jax
import jax.numpy as jnp
from jax import lax
from jax.experimental import pallas as pl
from jax.experimental.pallas import tpu as pltpu
from jax.experimental.pallas import tpu_sc as plsc

N = 10000
E = 320000
D = 128
DE = 16
H = 256
EPS = 1e-7
BN_EPS = 1e-5

DH = D // 2
C = 64
CP = C // 2
NTILES = 16
NCH = 320
EPT = NCH * C
E_PAD = NTILES * EPT
NP = 10112
RPT = NP // NTILES
BE = 512
NEB = E_PAD // BE
BN = 1000


def _edge_mm_body(ea_ref, we_ref, o_ref):
    o_ref[...] = jnp.dot(ea_ref[...], we_ref[0],
                         preferred_element_type=jnp.float32)


def _edge_mm(ea2, weblk):
    return pl.pallas_call(
        _edge_mm_body,
        grid=(2, NEB),
        in_specs=[
            pl.BlockSpec((BE // 2, 2 * DE), lambda c, i: (i, 0)),
            pl.BlockSpec((1, 2 * DE, D), lambda c, i: (c, 0, 0)),
        ],
        out_specs=pl.BlockSpec((BE // 2, D), lambda c, i: (c * NEB + i, 0)),
        out_shape=jax.ShapeDtypeStruct((E_PAD, D), jnp.float32),
    )(ea2, weblk)


def _split_body(x_ref, o_ref):
    xv = x_ref[...]
    z = jnp.zeros((BN, DH), jnp.float32)
    o_ref[0] = jnp.concatenate([xv[:, :DH], z], axis=1)
    o_ref[1] = jnp.concatenate([xv[:, DH:], z], axis=1)


def _split(x):
    out = pl.pallas_call(
        _split_body,
        grid=(N // BN,),
        in_specs=[pl.BlockSpec((BN, D), lambda i: (i, 0))],
        out_specs=pl.BlockSpec((2, BN, D), lambda i: (0, i, 0)),
        out_shape=jax.ShapeDtypeStruct((2, N, D), jnp.float32),
    )(x)
    return out.reshape(2 * N, D)


_sc_mesh = plsc.VectorSubcoreMesh(core_axis_name="c", subcore_axis_name="s",
                                  num_cores=2, num_subcores=16)


@functools.partial(
    pl.kernel,
    out_type=jax.ShapeDtypeStruct((2 * NP, D), jnp.float32),
    mesh=_sc_mesh,
    scratch_types=[
        pltpu.VMEM((4, 2 * C), jnp.int32),
        pltpu.VMEM((4, C), jnp.int32),
        pltpu.VMEM((2, C, D), jnp.float32),
        pltpu.VMEM((2, CP, D), jnp.float32),
        pltpu.VMEM((2, C, D), jnp.float32),
        pltpu.VMEM_SHARED((NP, D), jnp.float32),
        pltpu.SemaphoreType.DMA,
        pltpu.SemaphoreType.DMA,
        pltpu.SemaphoreType.DMA,
        pltpu.SemaphoreType.DMA,
        pltpu.SemaphoreType.DMA,
        pltpu.SemaphoreType.DMA,
        pltpu.SemaphoreType.DMA,
        pltpu.SemaphoreType.DMA,
        pltpu.SemaphoreType.DMA,
        pltpu.SemaphoreType.DMA,
    ],
)
def _sc_agg(x3, ep, sd_h, out2,
            sdv, dstv, gbuf, ebuf, obuf, acc,
            sg0, sg1, se0, se1, sv0, sv1, sv2, sv3, so0, so1):
    c = lax.axis_index("c")
    s = lax.axis_index("s")
    zero = jnp.zeros((16,), jnp.float32)
    nsplat = jnp.full((16,), N, jnp.int32)
    sg = (sg0, sg1)
    se = (se0, se1)
    sv = (sv0, sv1, sv2, sv3)
    so = (so0, so1)

    def zrow(i, carry):
        for j in range(D // 16):
            obuf[0, i, pl.ds(j * 16, 16)] = zero
            obuf[1, i, pl.ds(j * 16, 16)] = zero
        return carry

    lax.fori_loop(0, C, zrow, 0)
    for k in range(9):
        pltpu.sync_copy(obuf.at[0], acc.at[pl.ds(s * RPT + k * C, C)])
    pltpu.sync_copy(obuf.at[0, pl.ds(0, RPT - 576)],
                    acc.at[pl.ds(s * RPT + 576, RPT - 576)])
    for j in range(C // 16):
        for d in range(4):
            dstv[d, pl.ds(j * 16, 16)] = nsplat
    plsc.subcore_barrier()
    pltpu.async_copy(obuf.at[0], acc.at[dstv.at[2]], so[0], add=True)
    pltpu.async_copy(obuf.at[1], acc.at[dstv.at[3]], so[1], add=True)

    cN = c * N
    cEp = c * (E_PAD // 2)
    base0 = s * NCH

    def sd_issue_d(k, d):
        kk = jnp.minimum(k, NCH - 1)
        pltpu.async_copy(sd_h.at[pl.ds((base0 + kk) * 2 * C, 2 * C)],
                         sdv.at[d], sv[d])

    def sd_wait_d(k, d):
        kk = jnp.minimum(k, NCH - 1)
        pltpu.make_async_copy(sd_h.at[pl.ds((base0 + kk) * 2 * C, 2 * C)],
                              sdv.at[d], sv[d]).wait()
        for j in range(C // 16):
            dstv[d, pl.ds(j * 16, 16)] = sdv[d, pl.ds(C + j * 16, 16)]
            sdv[d, pl.ds(j * 16, 16)] = sdv[d, pl.ds(j * 16, 16)] + cN

    def ge_issue_db(k, d, b):
        kk = jnp.minimum(k, NCH - 1)
        pltpu.async_copy(x3.at[sdv.at[d, pl.ds(0, C)]], gbuf.at[b], sg[b])
        pltpu.async_copy(ep.at[pl.ds(cEp + (base0 + kk) * CP, CP)],
                         ebuf.at[b], se[b])

    def ge_wait_db(k, d, b):
        kk = jnp.minimum(k, NCH - 1)
        pltpu.make_async_copy(x3.at[sdv.at[d, pl.ds(0, C)]],
                              gbuf.at[b], sg[b]).wait()
        pltpu.make_async_copy(ep.at[pl.ds(cEp + (base0 + kk) * CP, CP)],
                              ebuf.at[b], se[b]).wait()

    def do_chunk(d, b):
        pltpu.make_async_copy(obuf.at[b], acc.at[dstv.at[d]], so[b]).wait()

        @plsc.parallel_loop(0, CP, step=1, unroll=2)
        def rowfn(r):
            r2 = 2 * r
            for j in range(DH // 16):
                sl = pl.ds(j * 16, 16)
                sl2 = pl.ds(DH + j * 16, 16)
                m0 = jnp.maximum(gbuf[b, r2, sl] + ebuf[b, r, sl], 0.0) + EPS
                t0 = jnp.exp(m0)
                obuf[b, r2, sl] = t0
                obuf[b, r2, sl2] = m0 * t0
                m1 = jnp.maximum(gbuf[b, r2 + 1, sl] + ebuf[b, r, sl2],
                                 0.0) + EPS
                t1 = jnp.exp(m1)
                obuf[b, r2 + 1, sl] = t1
                obuf[b, r2 + 1, sl2] = m1 * t1

        pltpu.async_copy(obuf.at[b], acc.at[dstv.at[d]], so[b], add=True)

    sd_issue_d(0, 0)
    sd_issue_d(1, 1)
    sd_wait_d(0, 0)
    ge_issue_db(0, 0, 0)

    def quad(q, carry):
        k0 = 4 * q
        for b4 in range(4):
            k = k0 + b4
            b = b4 % 2
            sd_wait_d(k + 1, (b4 + 1) % 4)
            ge_issue_db(k + 1, (b4 + 1) % 4, 1 - b)
            sd_issue_d(k + 2, (b4 + 2) % 4)
            ge_wait_db(k, b4 % 4, b)
            do_chunk(b4 % 4, b)
        return carry

    lax.fori_loop(0, NCH // 4, quad, 0)
    sd_wait_d(NCH, 1)
    ge_wait_db(NCH, 0, 0)
    pltpu.make_async_copy(obuf.at[0], acc.at[dstv.at[2]], so[0]).wait()
    pltpu.make_async_copy(obuf.at[1], acc.at[dstv.at[3]], so[1]).wait()
    plsc.subcore_barrier()

    for k in range(9):
        off = s * RPT + k * C
        pltpu.sync_copy(acc.at[pl.ds(off, C)], obuf.at[0])
        pltpu.sync_copy(obuf.at[0], out2.at[pl.ds(c * NP + off, C)])
    off = s * RPT + 576
    vb = obuf.at[0, pl.ds(0, RPT - 576)]
    pltpu.sync_copy(acc.at[pl.ds(off, RPT - 576)], vb)
    pltpu.sync_copy(vb, out2.at[pl.ds(c * NP + off, RPT - 576)])


def _node_mlp_body(relu_out, want_table, o2a_ref, o2b_ref, x_ref, wa_ref,
                   s1_ref, b1_ref, wb_ref, o_ref, t_ref):
    a = o2a_ref[0]
    b = o2b_ref[0]
    den = jnp.concatenate([a[:, :DH], b[:, :DH]], axis=1)
    num = jnp.concatenate([a[:, DH:], b[:, DH:]], axis=1)
    agg = num / jnp.where(den == 0.0, 1.0, den)
    o = agg + x_ref[...]
    h = jnp.dot(o, wa_ref[...], preferred_element_type=jnp.float32)
    h = jnp.maximum(h * s1_ref[...] + b1_ref[...], 0.0)
    y = jnp.dot(h, wb_ref[...], preferred_element_type=jnp.float32)
    if relu_out:
        y = jnp.maximum(y, 0.0)
    o_ref[...] = y
    if want_table:
        z = jnp.zeros((BN, DH), jnp.float32)
        t_ref[0] = jnp.concatenate([y[:, :DH], z], axis=1)
        t_ref[1] = jnp.concatenate([y[:, DH:], z], axis=1)


def _node_mlp(out2, x, Wa, s1, b1, Wb, relu_out, want_table):
    out2v = out2.reshape(2, NP, D)
    out_shapes = [jax.ShapeDtypeStruct((N, D), jnp.float32),
                  jax.ShapeDtypeStruct((2, N, D), jnp.float32)]
    y, tbl = pl.pallas_call(
        functools.partial(_node_mlp_body, relu_out, want_table),
        grid=(N // BN,),
        in_specs=[
            pl.BlockSpec((1, BN, D), lambda i: (0, i, 0)),
            pl.BlockSpec((1, BN, D), lambda i: (1, i, 0)),
            pl.BlockSpec((BN, D), lambda i: (i, 0)),
            pl.BlockSpec((D, H), lambda i: (0, 0)),
            pl.BlockSpec((1, H), lambda i: (0, 0)),
            pl.BlockSpec((1, H), lambda i: (0, 0)),
            pl.BlockSpec((H, D), lambda i: (0, 0)),
        ],
        out_specs=[pl.BlockSpec((BN, D), lambda i: (i, 0)),
                   pl.BlockSpec((2, BN, D), lambda i: (0, i, 0))],
        out_shape=out_shapes,
    )(out2v, out2v, x, Wa, s1, b1, Wb)
    return y, tbl.reshape(2 * N, D)


def _we_blockdiag(We):
    z = jnp.zeros((DE, DH), jnp.float32)
    blks = []
    for cc in range(2):
        wc = We[:, cc * DH:(cc + 1) * DH]
        blk = jnp.concatenate(
            [jnp.concatenate([wc, z], axis=1),
             jnp.concatenate([z, wc], axis=1)], axis=0)
        blks.append(blk)
    return jnp.stack(blks)


def _layer(x3, xin, sd, ea2, We, Wa, bnw, bnb, Wb, relu_out, want_table):
    ep = _edge_mm(ea2, _we_blockdiag(We))
    out2 = _sc_agg(x3, ep, sd)
    s1 = (bnw / jnp.sqrt(1.0 + BN_EPS)).reshape(1, H)
    b1 = bnb.reshape(1, H)
    return _node_mlp(out2, xin, Wa, s1, b1, Wb, relu_out, want_table)


def kernel(x, edge_index, edge_attr, We1, W1a, bn1w, bn1b, W1b,
           We2, W2a, bn2w, bn2b, W2b):
    pad = E_PAD - E
    src = jnp.concatenate([edge_index[0], jnp.zeros((pad,), jnp.int32)])
    dst = jnp.concatenate([edge_index[1], jnp.full((pad,), N, jnp.int32)])
    sd = jnp.concatenate(
        [src.reshape(-1, C), dst.reshape(-1, C)], axis=1).reshape(-1)
    ea2 = jnp.concatenate(
        [edge_attr, jnp.zeros((pad, DE), jnp.float32)],
        axis=0).reshape(E_PAD // 2, 2 * DE)
    x3 = _split(x)
    h, h3 = _layer(x3, x, sd, ea2, We1, W1a, bn1w, bn1b, W1b, True, True)
    y, _ = _layer(h3, h, sd, ea2, We2, W2a, bn2w, bn2b, W2b, False, False)
    return y


# speedup vs baseline: 1.5742x; 1.3404x over previous
import functools

import jax
import jax.numpy as jnp
from jax import lax
from jax.experimental import pallas as pl
from jax.experimental.pallas import tpu as pltpu
from jax.experimental.pallas import tpu_sc as plsc

N = 10000
E = 320000
D = 128
DE = 16
H = 256
EPS = 1e-7
BN_EPS = 1e-5

DH = D // 2
C = 64
CP = C // 2
NTILES = 16
NCH = 320
EPT = NCH * C
E_PAD = NTILES * EPT
NP = 10112
RPT = NP // NTILES
BE = 512
NEB = E_PAD // BE
BN = 1000


NEBH = (E_PAD // 2) // BE
NEAB = E // BE


def _edge_mm_body(ean_ref, eaf_ref, we_ref, o_ref):
    en = jnp.dot(ean_ref[...], we_ref[0], preferred_element_type=jnp.float32)
    ef = jnp.dot(eaf_ref[...], we_ref[0], preferred_element_type=jnp.float32)
    o_ref[...] = jnp.concatenate([en, ef], axis=1)


def _edge_mm(ea, we2):
    return pl.pallas_call(
        _edge_mm_body,
        grid=(2, NEBH),
        in_specs=[
            pl.BlockSpec((BE, DE), lambda c, i: (i, 0)),
            pl.BlockSpec((BE, DE),
                         lambda c, i: (jnp.minimum(NEBH + i, NEAB - 1), 0)),
            pl.BlockSpec((1, DE, DH), lambda c, i: (c, 0, 0)),
        ],
        out_specs=pl.BlockSpec((BE, D), lambda c, i: (c * NEBH + i, 0)),
        out_shape=jax.ShapeDtypeStruct((E_PAD, D), jnp.float32),
    )(ea, ea, we2)


NQ = E_PAD // 2 // 32
BQ = 80
FREAL = (E - E_PAD // 2) // 32


def _sd_body(sn_ref, sf_ref, dn_ref, df_ref, o_ref):
    q = pl.program_id(0)
    rid = q * BQ + jax.lax.broadcasted_iota(jnp.int32, (BQ, 32), 0)
    okf = rid < FREAL
    o_ref[...] = jnp.concatenate([
        sn_ref[0],
        jnp.where(okf, sf_ref[0], 0),
        dn_ref[0],
        jnp.where(okf, df_ref[0], N),
    ], axis=1)


def _sd_build(edge_index):
    ei = edge_index.reshape(2, E // 32, 32)
    nb = (E // 32) // BQ
    out = pl.pallas_call(
        _sd_body,
        grid=(NQ // BQ,),
        in_specs=[
            pl.BlockSpec((1, BQ, 32), lambda q: (0, q, 0)),
            pl.BlockSpec((1, BQ, 32),
                         lambda q: (0, jnp.minimum(NQ // BQ + q, nb - 1), 0)),
            pl.BlockSpec((1, BQ, 32), lambda q: (1, q, 0)),
            pl.BlockSpec((1, BQ, 32),
                         lambda q: (1, jnp.minimum(NQ // BQ + q, nb - 1), 0)),
        ],
        out_specs=pl.BlockSpec((BQ, 4 * 32), lambda q: (q, 0)),
        out_shape=jax.ShapeDtypeStruct((NQ, 4 * 32), jnp.int32),
    )(ei, ei, ei, ei)
    return out.reshape(-1)


def _split_body(x_ref, o_ref):
    xv = x_ref[...]
    z = jnp.zeros((BN, DH), jnp.float32)
    o_ref[0] = jnp.concatenate([xv[:, :DH], z], axis=1)
    o_ref[1] = jnp.concatenate([xv[:, DH:], z], axis=1)


def _split(x):
    out = pl.pallas_call(
        _split_body,
        grid=(N // BN,),
        in_specs=[pl.BlockSpec((BN, D), lambda i: (i, 0))],
        out_specs=pl.BlockSpec((2, BN, D), lambda i: (0, i, 0)),
        out_shape=jax.ShapeDtypeStruct((2, N, D), jnp.float32),
    )(x)
    return out.reshape(2 * N, D)


_sc_mesh = plsc.VectorSubcoreMesh(core_axis_name="c", subcore_axis_name="s",
                                  num_cores=2, num_subcores=16)


@functools.partial(
    pl.kernel,
    out_type=jax.ShapeDtypeStruct((2 * NP, D), jnp.float32),
    mesh=_sc_mesh,
    scratch_types=[
        pltpu.VMEM((4, 2 * C), jnp.int32),
        pltpu.VMEM((4, C), jnp.int32),
        pltpu.VMEM((2, C, D), jnp.float32),
        pltpu.VMEM((2, CP, D), jnp.float32),
        pltpu.VMEM((2, C, D), jnp.float32),
        pltpu.VMEM_SHARED((NP, D), jnp.float32),
        pltpu.SemaphoreType.DMA,
        pltpu.SemaphoreType.DMA,
        pltpu.SemaphoreType.DMA,
        pltpu.SemaphoreType.DMA,
        pltpu.SemaphoreType.DMA,
        pltpu.SemaphoreType.DMA,
        pltpu.SemaphoreType.DMA,
        pltpu.SemaphoreType.DMA,
        pltpu.SemaphoreType.DMA,
        pltpu.SemaphoreType.DMA,
    ],
)
def _sc_agg(x3, ep, sd_h, out2,
            sdv, dstv, gbuf, ebuf, obuf, acc,
            sg0, sg1, se0, se1, sv0, sv1, sv2, sv3, so0, so1):
    c = lax.axis_index("c")
    s = lax.axis_index("s")
    zero = jnp.zeros((16,), jnp.float32)
    nsplat = jnp.full((16,), N, jnp.int32)
    sg = (sg0, sg1)
    se = (se0, se1)
    sv = (sv0, sv1, sv2, sv3)
    so = (so0, so1)

    def zrow(i, carry):
        for j in range(D // 16):
            obuf[0, i, pl.ds(j * 16, 16)] = zero
            obuf[1, i, pl.ds(j * 16, 16)] = zero
        return carry

    lax.fori_loop(0, C, zrow, 0)
    for k in range(9):
        pltpu.sync_copy(obuf.at[0], acc.at[pl.ds(s * RPT + k * C, C)])
    pltpu.sync_copy(obuf.at[0, pl.ds(0, RPT - 576)],
                    acc.at[pl.ds(s * RPT + 576, RPT - 576)])
    for j in range(C // 16):
        for d in range(4):
            dstv[d, pl.ds(j * 16, 16)] = nsplat
    plsc.subcore_barrier()
    pltpu.async_copy(obuf.at[0], acc.at[dstv.at[2]], so[0], add=True)
    pltpu.async_copy(obuf.at[1], acc.at[dstv.at[3]], so[1], add=True)

    cN = c * N
    cEp = c * (E_PAD // 2)
    base0 = s * NCH

    def sd_issue_d(k, d):
        kk = jnp.minimum(k, NCH - 1)
        pltpu.async_copy(sd_h.at[pl.ds((base0 + kk) * 2 * C, 2 * C)],
                         sdv.at[d], sv[d])

    def sd_wait_d(k, d):
        kk = jnp.minimum(k, NCH - 1)
        pltpu.make_async_copy(sd_h.at[pl.ds((base0 + kk) * 2 * C, 2 * C)],
                              sdv.at[d], sv[d]).wait()
        for j in range(C // 16):
            dstv[d, pl.ds(j * 16, 16)] = sdv[d, pl.ds(C + j * 16, 16)]
            sdv[d, pl.ds(j * 16, 16)] = sdv[d, pl.ds(j * 16, 16)] + cN

    def ge_issue_db(k, d, b):
        kk = jnp.minimum(k, NCH - 1)
        pltpu.async_copy(x3.at[sdv.at[d, pl.ds(0, C)]], gbuf.at[b], sg[b])
        pltpu.async_copy(ep.at[pl.ds(cEp + (base0 + kk) * CP, CP)],
                         ebuf.at[b], se[b])

    def ge_wait_db(k, d, b):
        kk = jnp.minimum(k, NCH - 1)
        pltpu.make_async_copy(x3.at[sdv.at[d, pl.ds(0, C)]],
                              gbuf.at[b], sg[b]).wait()
        pltpu.make_async_copy(ep.at[pl.ds(cEp + (base0 + kk) * CP, CP)],
                              ebuf.at[b], se[b]).wait()

    def do_chunk(d, b):
        pltpu.make_async_copy(obuf.at[b], acc.at[dstv.at[d]], so[b]).wait()

        @plsc.parallel_loop(0, CP, step=1, unroll=2)
        def rowfn(r):
            rf = CP + r
            for j in range(DH // 16):
                sl = pl.ds(j * 16, 16)
                sl2 = pl.ds(DH + j * 16, 16)
                m0 = jnp.maximum(gbuf[b, r, sl] + ebuf[b, r, sl], 0.0) + EPS
                t0 = jnp.exp(m0)
                obuf[b, r, sl] = t0
                obuf[b, r, sl2] = m0 * t0
                m1 = jnp.maximum(gbuf[b, rf, sl] + ebuf[b, r, sl2], 0.0) + EPS
                t1 = jnp.exp(m1)
                obuf[b, rf, sl] = t1
                obuf[b, rf, sl2] = m1 * t1

        pltpu.async_copy(obuf.at[b], acc.at[dstv.at[d]], so[b], add=True)

    sd_issue_d(0, 0)
    sd_issue_d(1, 1)
    sd_wait_d(0, 0)
    ge_issue_db(0, 0, 0)

    def quad(q, carry):
        k0 = 4 * q
        for b4 in range(4):
            k = k0 + b4
            b = b4 % 2
            sd_wait_d(k + 1, (b4 + 1) % 4)
            ge_issue_db(k + 1, (b4 + 1) % 4, 1 - b)
            sd_issue_d(k + 2, (b4 + 2) % 4)
            ge_wait_db(k, b4 % 4, b)
            do_chunk(b4 % 4, b)
        return carry

    lax.fori_loop(0, NCH // 4, quad, 0)
    sd_wait_d(NCH, 1)
    ge_wait_db(NCH, 0, 0)
    pltpu.make_async_copy(obuf.at[0], acc.at[dstv.at[2]], so[0]).wait()
    pltpu.make_async_copy(obuf.at[1], acc.at[dstv.at[3]], so[1]).wait()
    plsc.subcore_barrier()

    for k in range(9):
        off = s * RPT + k * C
        pltpu.sync_copy(acc.at[pl.ds(off, C)], obuf.at[0])
        pltpu.sync_copy(obuf.at[0], out2.at[pl.ds(c * NP + off, C)])
    off = s * RPT + 576
    vb = obuf.at[0, pl.ds(0, RPT - 576)]
    pltpu.sync_copy(acc.at[pl.ds(off, RPT - 576)], vb)
    pltpu.sync_copy(vb, out2.at[pl.ds(c * NP + off, RPT - 576)])


def _node_mlp_body(relu_out, want_table, o2a_ref, o2b_ref, x_ref, wa_ref,
                   s1_ref, b1_ref, wb_ref, o_ref, t_ref):
    a = o2a_ref[0]
    b = o2b_ref[0]
    den = jnp.concatenate([a[:, :DH], b[:, :DH]], axis=1)
    num = jnp.concatenate([a[:, DH:], b[:, DH:]], axis=1)
    agg = num / jnp.where(den == 0.0, 1.0, den)
    o = agg + x_ref[...]
    h = jnp.dot(o, wa_ref[...], preferred_element_type=jnp.float32)
    h = jnp.maximum(h * s1_ref[...] + b1_ref[...], 0.0)
    y = jnp.dot(h, wb_ref[...], preferred_element_type=jnp.float32)
    if relu_out:
        y = jnp.maximum(y, 0.0)
    o_ref[...] = y
    if want_table:
        z = jnp.zeros((BN, DH), jnp.float32)
        t_ref[0] = jnp.concatenate([y[:, :DH], z], axis=1)
        t_ref[1] = jnp.concatenate([y[:, DH:], z], axis=1)


def _node_mlp(out2, x, Wa, s1, b1, Wb, relu_out, want_table):
    out2v = out2.reshape(2, NP, D)
    out_shapes = [jax.ShapeDtypeStruct((N, D), jnp.float32),
                  jax.ShapeDtypeStruct((2, N, D), jnp.float32)]
    y, tbl = pl.pallas_call(
        functools.partial(_node_mlp_body, relu_out, want_table),
        grid=(N // BN,),
        in_specs=[
            pl.BlockSpec((1, BN, D), lambda i: (0, i, 0)),
            pl.BlockSpec((1, BN, D), lambda i: (1, i, 0)),
            pl.BlockSpec((BN, D), lambda i: (i, 0)),
            pl.BlockSpec((D, H), lambda i: (0, 0)),
            pl.BlockSpec((1, H), lambda i: (0, 0)),
            pl.BlockSpec((1, H), lambda i: (0, 0)),
            pl.BlockSpec((H, D), lambda i: (0, 0)),
        ],
        out_specs=[pl.BlockSpec((BN, D), lambda i: (i, 0)),
                   pl.BlockSpec((2, BN, D), lambda i: (0, i, 0))],
        out_shape=out_shapes,
    )(out2v, out2v, x, Wa, s1, b1, Wb)
    return y, tbl.reshape(2 * N, D)


def _layer(x3, xin, sd, ea, We, Wa, bnw, bnb, Wb, relu_out, want_table):
    we2 = We.reshape(DE, 2, DH).transpose(1, 0, 2)
    ep = _edge_mm(ea, we2)
    out2 = _sc_agg(x3, ep, sd)
    s1 = (bnw / jnp.sqrt(1.0 + BN_EPS)).reshape(1, H)
    b1 = bnb.reshape(1, H)
    return _node_mlp(out2, xin, Wa, s1, b1, Wb, relu_out, want_table)


def kernel(x, edge_index, edge_attr, We1, W1a, bn1w, bn1b, W1b,
           We2, W2a, bn2w, bn2b, W2b):
    sd = _sd_build(edge_index)
    x3 = _split(x)
    h, h3 = _layer(x3, x, sd, edge_attr, We1, W1a, bn1w, bn1b, W1b, True, True)
    y, _ = _layer(h3, h, sd, edge_attr, We2, W2a, bn2w, bn2b, W2b, False, False)
    return y


# speedup vs baseline: 1.8225x; 1.1577x over previous
import functools

import jax
import jax.numpy as jnp
from jax import lax
from jax.experimental import pallas as pl
from jax.experimental.pallas import tpu as pltpu
from jax.experimental.pallas import tpu_sc as plsc

N = 10000
E = 320000
D = 128
DE = 16
H = 256
EPS = 1e-7
BN_EPS = 1e-5

DH = D // 2
C = 64
CP = C // 2
NTILES = 16
NCH = 320
EPT = NCH * C
E_PAD = NTILES * EPT
NP = 10112
RPT = NP // NTILES
BE = 2560
BN = 1000


NEBH = (E_PAD // 2) // BE
NEAB = E // BE


def _edge_mm_body(ean_ref, eaf_ref, we_ref, o_ref):
    en = jnp.dot(ean_ref[...], we_ref[0], preferred_element_type=jnp.float32)
    ef = jnp.dot(eaf_ref[...], we_ref[0], preferred_element_type=jnp.float32)
    o_ref[...] = jnp.concatenate([en, ef], axis=1)


def _edge_mm(ea, we2):
    return pl.pallas_call(
        _edge_mm_body,
        grid=(2, NEBH),
        in_specs=[
            pl.BlockSpec((BE, DE), lambda c, i: (i, 0)),
            pl.BlockSpec((BE, DE),
                         lambda c, i: (jnp.minimum(NEBH + i, NEAB - 1), 0)),
            pl.BlockSpec((1, DE, DH), lambda c, i: (c, 0, 0)),
        ],
        out_specs=pl.BlockSpec((BE, D), lambda c, i: (c * NEBH + i, 0)),
        out_shape=jax.ShapeDtypeStruct((E_PAD, D), jnp.float32),
    )(ea, ea, we2)


NQ = E_PAD // 2 // 32
BQ = 80
FREAL = (E - E_PAD // 2) // 32


def _sd_body(sn_ref, sf_ref, dn_ref, df_ref, o_ref):
    q = pl.program_id(0)
    rid = q * BQ + jax.lax.broadcasted_iota(jnp.int32, (BQ, 32), 0)
    okf = rid < FREAL
    o_ref[...] = jnp.concatenate([
        sn_ref[0],
        jnp.where(okf, sf_ref[0], 0),
        dn_ref[0],
        jnp.where(okf, df_ref[0], N),
    ], axis=1)


def _sd_build(edge_index):
    ei = edge_index.reshape(2, E // 32, 32)
    nb = (E // 32) // BQ
    out = pl.pallas_call(
        _sd_body,
        grid=(NQ // BQ,),
        in_specs=[
            pl.BlockSpec((1, BQ, 32), lambda q: (0, q, 0)),
            pl.BlockSpec((1, BQ, 32),
                         lambda q: (0, jnp.minimum(NQ // BQ + q, nb - 1), 0)),
            pl.BlockSpec((1, BQ, 32), lambda q: (1, q, 0)),
            pl.BlockSpec((1, BQ, 32),
                         lambda q: (1, jnp.minimum(NQ // BQ + q, nb - 1), 0)),
        ],
        out_specs=pl.BlockSpec((BQ, 4 * 32), lambda q: (q, 0)),
        out_shape=jax.ShapeDtypeStruct((NQ, 4 * 32), jnp.int32),
    )(ei, ei, ei, ei)
    return out.reshape(-1)


def _split_body(x_ref, o_ref):
    xv = x_ref[...]
    z = jnp.zeros((BN, DH), jnp.float32)
    o_ref[0] = jnp.concatenate([xv[:, :DH], z], axis=1)
    o_ref[1] = jnp.concatenate([xv[:, DH:], z], axis=1)


def _split(x):
    out = pl.pallas_call(
        _split_body,
        grid=(N // BN,),
        in_specs=[pl.BlockSpec((BN, D), lambda i: (i, 0))],
        out_specs=pl.BlockSpec((2, BN, D), lambda i: (0, i, 0)),
        out_shape=jax.ShapeDtypeStruct((2, N, D), jnp.float32),
    )(x)
    return out.reshape(2 * N, D)


_sc_mesh = plsc.VectorSubcoreMesh(core_axis_name="c", subcore_axis_name="s",
                                  num_cores=2, num_subcores=16)


@functools.partial(
    pl.kernel,
    out_type=jax.ShapeDtypeStruct((2 * NP, D), jnp.float32),
    mesh=_sc_mesh,
    scratch_types=[
        pltpu.VMEM((4, 2 * C), jnp.int32),
        pltpu.VMEM((4, C), jnp.int32),
        pltpu.VMEM((2, C, D), jnp.float32),
        pltpu.VMEM((2, CP, D), jnp.float32),
        pltpu.VMEM((2, C, D), jnp.float32),
        pltpu.VMEM_SHARED((NP, D), jnp.float32),
        pltpu.SemaphoreType.DMA,
        pltpu.SemaphoreType.DMA,
        pltpu.SemaphoreType.DMA,
        pltpu.SemaphoreType.DMA,
        pltpu.SemaphoreType.DMA,
        pltpu.SemaphoreType.DMA,
        pltpu.SemaphoreType.DMA,
        pltpu.SemaphoreType.DMA,
        pltpu.SemaphoreType.DMA,
        pltpu.SemaphoreType.DMA,
    ],
)
def _sc_agg(x3, ep, sd_h, out2,
            sdv, dstv, gbuf, ebuf, obuf, acc,
            sg0, sg1, se0, se1, sv0, sv1, sv2, sv3, so0, so1):
    c = lax.axis_index("c")
    s = lax.axis_index("s")
    zero = jnp.zeros((16,), jnp.float32)
    nsplat = jnp.full((16,), N, jnp.int32)
    sg = (sg0, sg1)
    se = (se0, se1)
    sv = (sv0, sv1, sv2, sv3)
    so = (so0, so1)

    def zrow(i, carry):
        for j in range(D // 16):
            obuf[0, i, pl.ds(j * 16, 16)] = zero
            obuf[1, i, pl.ds(j * 16, 16)] = zero
        return carry

    lax.fori_loop(0, C, zrow, 0)
    for k in range(9):
        pltpu.sync_copy(obuf.at[0], acc.at[pl.ds(s * RPT + k * C, C)])
    pltpu.sync_copy(obuf.at[0, pl.ds(0, RPT - 576)],
                    acc.at[pl.ds(s * RPT + 576, RPT - 576)])
    for j in range(C // 16):
        for d in range(4):
            dstv[d, pl.ds(j * 16, 16)] = nsplat
    plsc.subcore_barrier()
    pltpu.async_copy(obuf.at[0], acc.at[dstv.at[2]], so[0], add=True)
    pltpu.async_copy(obuf.at[1], acc.at[dstv.at[3]], so[1], add=True)

    cN = c * N
    cEp = c * (E_PAD // 2)
    base0 = s * NCH

    def sd_issue_d(k, d):
        kk = jnp.minimum(k, NCH - 1)
        pltpu.async_copy(sd_h.at[pl.ds((base0 + kk) * 2 * C, 2 * C)],
                         sdv.at[d], sv[d])

    def sd_wait_d(k, d):
        kk = jnp.minimum(k, NCH - 1)
        pltpu.make_async_copy(sd_h.at[pl.ds((base0 + kk) * 2 * C, 2 * C)],
                              sdv.at[d], sv[d]).wait()
        for j in range(C // 16):
            dstv[d, pl.ds(j * 16, 16)] = sdv[d, pl.ds(C + j * 16, 16)]
            sdv[d, pl.ds(j * 16, 16)] = sdv[d, pl.ds(j * 16, 16)] + cN

    def ge_issue_db(k, d, b):
        kk = jnp.minimum(k, NCH - 1)
        pltpu.async_copy(x3.at[sdv.at[d, pl.ds(0, C)]], gbuf.at[b], sg[b])
        pltpu.async_copy(ep.at[pl.ds(cEp + (base0 + kk) * CP, CP)],
                         ebuf.at[b], se[b])

    def ge_wait_db(k, d, b):
        kk = jnp.minimum(k, NCH - 1)
        pltpu.make_async_copy(x3.at[sdv.at[d, pl.ds(0, C)]],
                              gbuf.at[b], sg[b]).wait()
        pltpu.make_async_copy(ep.at[pl.ds(cEp + (base0 + kk) * CP, CP)],
                              ebuf.at[b], se[b]).wait()

    def do_chunk(d, b):
        pltpu.make_async_copy(obuf.at[b], acc.at[dstv.at[d]], so[b]).wait()

        @plsc.parallel_loop(0, CP, step=1, unroll=4)
        def rowfn(r):
            rf = CP + r
            for j in range(DH // 16):
                sl = pl.ds(j * 16, 16)
                sl2 = pl.ds(DH + j * 16, 16)
                m0 = jnp.maximum(gbuf[b, r, sl] + ebuf[b, r, sl], 0.0) + EPS
                t0 = jnp.exp(m0)
                obuf[b, r, sl] = t0
                obuf[b, r, sl2] = m0 * t0
                m1 = jnp.maximum(gbuf[b, rf, sl] + ebuf[b, r, sl2], 0.0) + EPS
                t1 = jnp.exp(m1)
                obuf[b, rf, sl] = t1
                obuf[b, rf, sl2] = m1 * t1

        pltpu.async_copy(obuf.at[b], acc.at[dstv.at[d]], so[b], add=True)

    sd_issue_d(0, 0)
    sd_issue_d(1, 1)
    sd_wait_d(0, 0)
    ge_issue_db(0, 0, 0)

    def quad(q, carry):
        k0 = 4 * q
        for b4 in range(4):
            k = k0 + b4
            b = b4 % 2
            sd_wait_d(k + 1, (b4 + 1) % 4)
            ge_issue_db(k + 1, (b4 + 1) % 4, 1 - b)
            sd_issue_d(k + 2, (b4 + 2) % 4)
            ge_wait_db(k, b4 % 4, b)
            do_chunk(b4 % 4, b)
        return carry

    lax.fori_loop(0, NCH // 4, quad, 0)
    sd_wait_d(NCH, 1)
    ge_wait_db(NCH, 0, 0)
    pltpu.make_async_copy(obuf.at[0], acc.at[dstv.at[2]], so[0]).wait()
    pltpu.make_async_copy(obuf.at[1], acc.at[dstv.at[3]], so[1]).wait()
    plsc.subcore_barrier()

    for k in range(9):
        off = s * RPT + k * C
        pltpu.sync_copy(acc.at[pl.ds(off, C)], obuf.at[0])
        pltpu.sync_copy(obuf.at[0], out2.at[pl.ds(c * NP + off, C)])
    off = s * RPT + 576
    vb = obuf.at[0, pl.ds(0, RPT - 576)]
    pltpu.sync_copy(acc.at[pl.ds(off, RPT - 576)], vb)
    pltpu.sync_copy(vb, out2.at[pl.ds(c * NP + off, RPT - 576)])


def _node_mlp_body(relu_out, want_table, o2a_ref, o2b_ref, x_ref, wa_ref,
                   s1_ref, b1_ref, wb_ref, o_ref, t_ref):
    a = o2a_ref[0]
    b = o2b_ref[0]
    den = jnp.concatenate([a[:, :DH], b[:, :DH]], axis=1)
    num = jnp.concatenate([a[:, DH:], b[:, DH:]], axis=1)
    agg = num / jnp.where(den == 0.0, 1.0, den)
    o = agg + x_ref[...]
    h = jnp.dot(o, wa_ref[...], preferred_element_type=jnp.float32)
    h = jnp.maximum(h * s1_ref[...] + b1_ref[...], 0.0)
    y = jnp.dot(h, wb_ref[...], preferred_element_type=jnp.float32)
    if relu_out:
        y = jnp.maximum(y, 0.0)
    o_ref[...] = y
    if want_table:
        z = jnp.zeros((BN, DH), jnp.float32)
        t_ref[0] = jnp.concatenate([y[:, :DH], z], axis=1)
        t_ref[1] = jnp.concatenate([y[:, DH:], z], axis=1)


def _node_mlp(out2, x, Wa, s1, b1, Wb, relu_out, want_table):
    out2v = out2.reshape(2, NP, D)
    out_shapes = [jax.ShapeDtypeStruct((N, D), jnp.float32),
                  jax.ShapeDtypeStruct((2, N, D), jnp.float32)]
    y, tbl = pl.pallas_call(
        functools.partial(_node_mlp_body, relu_out, want_table),
        grid=(N // BN,),
        in_specs=[
            pl.BlockSpec((1, BN, D), lambda i: (0, i, 0)),
            pl.BlockSpec((1, BN, D), lambda i: (1, i, 0)),
            pl.BlockSpec((BN, D), lambda i: (i, 0)),
            pl.BlockSpec((D, H), lambda i: (0, 0)),
            pl.BlockSpec((1, H), lambda i: (0, 0)),
            pl.BlockSpec((1, H), lambda i: (0, 0)),
            pl.BlockSpec((H, D), lambda i: (0, 0)),
        ],
        out_specs=[pl.BlockSpec((BN, D), lambda i: (i, 0)),
                   pl.BlockSpec((2, BN, D), lambda i: (0, i, 0))],
        out_shape=out_shapes,
    )(out2v, out2v, x, Wa, s1, b1, Wb)
    return y, tbl.reshape(2 * N, D)


def _layer(x3, xin, sd, ea, We, Wa, bnw, bnb, Wb, relu_out, want_table):
    we2 = We.reshape(DE, 2, DH).transpose(1, 0, 2)
    ep = _edge_mm(ea, we2)
    out2 = _sc_agg(x3, ep, sd)
    s1 = (bnw / jnp.sqrt(1.0 + BN_EPS)).reshape(1, H)
    b1 = bnb.reshape(1, H)
    return _node_mlp(out2, xin, Wa, s1, b1, Wb, relu_out, want_table)


def kernel(x, edge_index, edge_attr, We1, W1a, bn1w, bn1b, W1b,
           We2, W2a, bn2w, bn2b, W2b):
    sd = _sd_build(edge_index)
    x3 = _split(x)
    h, h3 = _layer(x3, x, sd, edge_attr, We1, W1a, bn1w, bn1b, W1b, True, True)
    y, _ = _layer(h3, h, sd, edge_attr, We2, W2a, bn2w, bn2b, W2b, False, False)
    return y


# speedup vs baseline: 1.9798x; 1.0863x over previous
import functools

import jax
import jax.numpy as jnp
from jax import lax
from jax.experimental import pallas as pl
from jax.experimental.pallas import tpu as pltpu
from jax.experimental.pallas import tpu_sc as plsc

N = 10000
E = 320000
D = 128
DE = 16
H = 256
EPS = 1e-7
BN_EPS = 1e-5

DH = D // 2
C = 64
CP = C // 2
NTILES = 16
NCH = 320
EPT = NCH * C
E_PAD = NTILES * EPT
NP = 10112
RPT = NP // NTILES
BE = 2560
BN = 1000


NEBH = (E_PAD // 2) // BE
NEAB = E // BE


def _edge_mm_body(ean_ref, eaf_ref, we_ref, o_ref):
    ean = ean_ref[...]
    eaf = eaf_ref[...]
    for cc in range(2):
        en = jnp.dot(ean, we_ref[cc], preferred_element_type=jnp.float32)
        ef = jnp.dot(eaf, we_ref[cc], preferred_element_type=jnp.float32)
        o_ref[cc] = jnp.concatenate([en, ef], axis=1)


def _edge_mm(ea, we2):
    out = pl.pallas_call(
        _edge_mm_body,
        grid=(NEBH,),
        in_specs=[
            pl.BlockSpec((BE, DE), lambda i: (i, 0)),
            pl.BlockSpec((BE, DE),
                         lambda i: (jnp.minimum(NEBH + i, NEAB - 1), 0)),
            pl.BlockSpec((2, DE, DH), lambda i: (0, 0, 0)),
        ],
        out_specs=pl.BlockSpec((2, BE, D), lambda i: (0, i, 0)),
        out_shape=jax.ShapeDtypeStruct((2, E_PAD // 2, D), jnp.float32),
    )(ea, ea, we2)
    return out.reshape(E_PAD, D)


NQ = E_PAD // 2 // 32
BQ = 80
FREAL = (E - E_PAD // 2) // 32


def _sd_body(sn_ref, sf_ref, dn_ref, df_ref, o_ref):
    q = pl.program_id(0)
    rid = q * BQ + jax.lax.broadcasted_iota(jnp.int32, (BQ, 32), 0)
    okf = rid < FREAL
    o_ref[...] = jnp.concatenate([
        sn_ref[0],
        jnp.where(okf, sf_ref[0], 0),
        dn_ref[0],
        jnp.where(okf, df_ref[0], N),
    ], axis=1)


def _sd_build(edge_index):
    ei = edge_index.reshape(2, E // 32, 32)
    nb = (E // 32) // BQ
    out = pl.pallas_call(
        _sd_body,
        grid=(NQ // BQ,),
        in_specs=[
            pl.BlockSpec((1, BQ, 32), lambda q: (0, q, 0)),
            pl.BlockSpec((1, BQ, 32),
                         lambda q: (0, jnp.minimum(NQ // BQ + q, nb - 1), 0)),
            pl.BlockSpec((1, BQ, 32), lambda q: (1, q, 0)),
            pl.BlockSpec((1, BQ, 32),
                         lambda q: (1, jnp.minimum(NQ // BQ + q, nb - 1), 0)),
        ],
        out_specs=pl.BlockSpec((BQ, 4 * 32), lambda q: (q, 0)),
        out_shape=jax.ShapeDtypeStruct((NQ, 4 * 32), jnp.int32),
    )(ei, ei, ei, ei)
    return out.reshape(-1)


def _split_body(x_ref, o_ref):
    xv = x_ref[...]
    z = jnp.zeros((BN, DH), jnp.float32)
    o_ref[0] = jnp.concatenate([xv[:, :DH], z], axis=1)
    o_ref[1] = jnp.concatenate([xv[:, DH:], z], axis=1)


def _split(x):
    out = pl.pallas_call(
        _split_body,
        grid=(N // BN,),
        in_specs=[pl.BlockSpec((BN, D), lambda i: (i, 0))],
        out_specs=pl.BlockSpec((2, BN, D), lambda i: (0, i, 0)),
        out_shape=jax.ShapeDtypeStruct((2, N, D), jnp.float32),
    )(x)
    return out.reshape(2 * N, D)


_sc_mesh = plsc.VectorSubcoreMesh(core_axis_name="c", subcore_axis_name="s",
                                  num_cores=2, num_subcores=16)


@functools.partial(
    pl.kernel,
    out_type=jax.ShapeDtypeStruct((2 * NP, D), jnp.float32),
    mesh=_sc_mesh,
    scratch_types=[
        pltpu.VMEM((4, 2 * C), jnp.int32),
        pltpu.VMEM((4, C), jnp.int32),
        pltpu.VMEM((2, C, D), jnp.float32),
        pltpu.VMEM((2, CP, D), jnp.float32),
        pltpu.VMEM((2, C, D), jnp.float32),
        pltpu.VMEM_SHARED((NP, D), jnp.float32),
        pltpu.SemaphoreType.DMA,
        pltpu.SemaphoreType.DMA,
        pltpu.SemaphoreType.DMA,
        pltpu.SemaphoreType.DMA,
        pltpu.SemaphoreType.DMA,
        pltpu.SemaphoreType.DMA,
        pltpu.SemaphoreType.DMA,
        pltpu.SemaphoreType.DMA,
        pltpu.SemaphoreType.DMA,
        pltpu.SemaphoreType.DMA,
    ],
)
def _sc_agg(x3, ep, sd_h, out2,
            sdv, dstv, gbuf, ebuf, obuf, acc,
            sg0, sg1, se0, se1, sv0, sv1, sv2, sv3, so0, so1):
    c = lax.axis_index("c")
    s = lax.axis_index("s")
    zero = jnp.zeros((16,), jnp.float32)
    nsplat = jnp.full((16,), N, jnp.int32)
    sg = (sg0, sg1)
    se = (se0, se1)
    sv = (sv0, sv1, sv2, sv3)
    so = (so0, so1)

    def zrow(i, carry):
        for j in range(D // 16):
            obuf[0, i, pl.ds(j * 16, 16)] = zero
            obuf[1, i, pl.ds(j * 16, 16)] = zero
        return carry

    lax.fori_loop(0, C, zrow, 0)
    for k in range(9):
        pltpu.sync_copy(obuf.at[0], acc.at[pl.ds(s * RPT + k * C, C)])
    pltpu.sync_copy(obuf.at[0, pl.ds(0, RPT - 576)],
                    acc.at[pl.ds(s * RPT + 576, RPT - 576)])
    for j in range(C // 16):
        for d in range(4):
            dstv[d, pl.ds(j * 16, 16)] = nsplat
    plsc.subcore_barrier()
    pltpu.async_copy(obuf.at[0], acc.at[dstv.at[2]], so[0], add=True)
    pltpu.async_copy(obuf.at[1], acc.at[dstv.at[3]], so[1], add=True)

    cN = c * N
    cEp = c * (E_PAD // 2)
    base0 = s * NCH

    def sd_issue_d(k, d):
        kk = jnp.minimum(k, NCH - 1)
        pltpu.async_copy(sd_h.at[pl.ds((base0 + kk) * 2 * C, 2 * C)],
                         sdv.at[d], sv[d])

    def sd_wait_d(k, d):
        kk = jnp.minimum(k, NCH - 1)
        pltpu.make_async_copy(sd_h.at[pl.ds((base0 + kk) * 2 * C, 2 * C)],
                              sdv.at[d], sv[d]).wait()
        for j in range(C // 16):
            dstv[d, pl.ds(j * 16, 16)] = sdv[d, pl.ds(C + j * 16, 16)]
            sdv[d, pl.ds(j * 16, 16)] = sdv[d, pl.ds(j * 16, 16)] + cN

    def ge_issue_db(k, d, b):
        kk = jnp.minimum(k, NCH - 1)
        pltpu.async_copy(x3.at[sdv.at[d, pl.ds(0, C)]], gbuf.at[b], sg[b])
        pltpu.async_copy(ep.at[pl.ds(cEp + (base0 + kk) * CP, CP)],
                         ebuf.at[b], se[b])

    def ge_wait_db(k, d, b):
        kk = jnp.minimum(k, NCH - 1)
        pltpu.make_async_copy(x3.at[sdv.at[d, pl.ds(0, C)]],
                              gbuf.at[b], sg[b]).wait()
        pltpu.make_async_copy(ep.at[pl.ds(cEp + (base0 + kk) * CP, CP)],
                              ebuf.at[b], se[b]).wait()

    def do_chunk(d, b):
        pltpu.make_async_copy(obuf.at[b], acc.at[dstv.at[d]], so[b]).wait()

        @plsc.parallel_loop(0, CP, step=1, unroll=4)
        def rowfn(r):
            rf = CP + r
            for j in range(DH // 16):
                sl = pl.ds(j * 16, 16)
                sl2 = pl.ds(DH + j * 16, 16)
                m0 = jnp.maximum(gbuf[b, r, sl] + ebuf[b, r, sl], 0.0) + EPS
                t0 = jnp.exp(m0)
                obuf[b, r, sl] = t0
                obuf[b, r, sl2] = m0 * t0
                m1 = jnp.maximum(gbuf[b, rf, sl] + ebuf[b, r, sl2], 0.0) + EPS
                t1 = jnp.exp(m1)
                obuf[b, rf, sl] = t1
                obuf[b, rf, sl2] = m1 * t1

        pltpu.async_copy(obuf.at[b], acc.at[dstv.at[d]], so[b], add=True)

    sd_issue_d(0, 0)
    sd_issue_d(1, 1)
    sd_wait_d(0, 0)
    ge_issue_db(0, 0, 0)

    def quad(q, carry):
        k0 = 4 * q
        for b4 in range(4):
            k = k0 + b4
            b = b4 % 2
            sd_wait_d(k + 1, (b4 + 1) % 4)
            ge_issue_db(k + 1, (b4 + 1) % 4, 1 - b)
            sd_issue_d(k + 2, (b4 + 2) % 4)
            ge_wait_db(k, b4 % 4, b)
            do_chunk(b4 % 4, b)
        return carry

    lax.fori_loop(0, NCH // 4, quad, 0)
    sd_wait_d(NCH, 1)
    ge_wait_db(NCH, 0, 0)
    pltpu.make_async_copy(obuf.at[0], acc.at[dstv.at[2]], so[0]).wait()
    pltpu.make_async_copy(obuf.at[1], acc.at[dstv.at[3]], so[1]).wait()
    plsc.subcore_barrier()

    for k in range(9):
        off = s * RPT + k * C
        pltpu.sync_copy(acc.at[pl.ds(off, C)], obuf.at[0])
        pltpu.sync_copy(obuf.at[0], out2.at[pl.ds(c * NP + off, C)])
    off = s * RPT + 576
    vb = obuf.at[0, pl.ds(0, RPT - 576)]
    pltpu.sync_copy(acc.at[pl.ds(off, RPT - 576)], vb)
    pltpu.sync_copy(vb, out2.at[pl.ds(c * NP + off, RPT - 576)])


def _node_mlp_body(relu_out, want_table, o2a_ref, o2b_ref, x_ref, wa_ref,
                   s1_ref, b1_ref, wb_ref, o_ref, t_ref):
    a = o2a_ref[0]
    b = o2b_ref[0]
    den = jnp.concatenate([a[:, :DH], b[:, :DH]], axis=1)
    num = jnp.concatenate([a[:, DH:], b[:, DH:]], axis=1)
    agg = num / jnp.where(den == 0.0, 1.0, den)
    o = agg + x_ref[...]
    h = jnp.dot(o, wa_ref[...], preferred_element_type=jnp.float32)
    h = jnp.maximum(h * s1_ref[...] + b1_ref[...], 0.0)
    y = jnp.dot(h, wb_ref[...], preferred_element_type=jnp.float32)
    if relu_out:
        y = jnp.maximum(y, 0.0)
    o_ref[...] = y
    if want_table:
        z = jnp.zeros((BN, DH), jnp.float32)
        t_ref[0] = jnp.concatenate([y[:, :DH], z], axis=1)
        t_ref[1] = jnp.concatenate([y[:, DH:], z], axis=1)


def _node_mlp(out2, x, Wa, s1, b1, Wb, relu_out, want_table):
    out2v = out2.reshape(2, NP, D)
    out_shapes = [jax.ShapeDtypeStruct((N, D), jnp.float32),
                  jax.ShapeDtypeStruct((2, N, D), jnp.float32)]
    y, tbl = pl.pallas_call(
        functools.partial(_node_mlp_body, relu_out, want_table),
        grid=(N // BN,),
        in_specs=[
            pl.BlockSpec((1, BN, D), lambda i: (0, i, 0)),
            pl.BlockSpec((1, BN, D), lambda i: (1, i, 0)),
            pl.BlockSpec((BN, D), lambda i: (i, 0)),
            pl.BlockSpec((D, H), lambda i: (0, 0)),
            pl.BlockSpec((1, H), lambda i: (0, 0)),
            pl.BlockSpec((1, H), lambda i: (0, 0)),
            pl.BlockSpec((H, D), lambda i: (0, 0)),
        ],
        out_specs=[pl.BlockSpec((BN, D), lambda i: (i, 0)),
                   pl.BlockSpec((2, BN, D), lambda i: (0, i, 0))],
        out_shape=out_shapes,
    )(out2v, out2v, x, Wa, s1, b1, Wb)
    return y, tbl.reshape(2 * N, D)


def _layer(x3, xin, sd, ea, We, Wa, bnw, bnb, Wb, relu_out, want_table):
    we2 = We.reshape(DE, 2, DH).transpose(1, 0, 2)
    ep = _edge_mm(ea, we2)
    out2 = _sc_agg(x3, ep, sd)
    s1 = (bnw / jnp.sqrt(1.0 + BN_EPS)).reshape(1, H)
    b1 = bnb.reshape(1, H)
    return _node_mlp(out2, xin, Wa, s1, b1, Wb, relu_out, want_table)


def kernel(x, edge_index, edge_attr, We1, W1a, bn1w, bn1b, W1b,
           We2, W2a, bn2w, bn2b, W2b):
    sd = _sd_build(edge_index)
    x3 = _split(x)
    h, h3 = _layer(x3, x, sd, edge_attr, We1, W1a, bn1w, bn1b, W1b, True, True)
    y, _ = _layer(h3, h, sd, edge_attr, We2, W2a, bn2w, bn2b, W2b, False, False)
    return y
